# baseline (speedup 1.0000x reference)
import jax
import jax.numpy as jnp
from jax.experimental import pallas as pl

_ROWS = 2 * 8192
_COLS = 4096
_BLOCK_ROWS = 256


def _copy_body(i_ref, o_ref):
    o_ref[...] = i_ref[...]


def kernel(x, bit, alpha):
    del bit, alpha
    x2 = x.reshape(_ROWS, _COLS)
    out = pl.pallas_call(
        _copy_body,
        grid=(_ROWS // _BLOCK_ROWS,),
        in_specs=[pl.BlockSpec((_BLOCK_ROWS, _COLS), lambda i: (i, 0))],
        out_specs=pl.BlockSpec((_BLOCK_ROWS, _COLS), lambda i: (i, 0)),
        out_shape=jax.ShapeDtypeStruct((_ROWS, _COLS), x.dtype),
    )(x2)
    return out.reshape(x.shape)
